# baseline (speedup 1.0000x reference)
import jax
import jax.numpy as jnp
from jax import lax
from jax.experimental import pallas as pl
from jax.experimental.pallas import tpu as pltpu
from jax.experimental.pallas import tpu_sc as plsc

NC = 2
NS = 16
NW = NC * NS

IDX_PER_STREAM = 128
CHUNK = 32
NBUF = 3


def _gcd(a, b):
    while b:
        a, b = b, a % b
    return a


def _make_sc_call(B, S, L, D, V):
    SENT = B * S
    assert SENT % NW == 0
    sent_per_w = SENT // NW
    chunk = CHUNK
    ipc = chunk * L
    assert ipc % IDX_PER_STREAM == 0 and IDX_PER_STREAM % 8 == 0
    n_streams = ipc // IDX_PER_STREAM
    assert sent_per_w % chunk == 0
    n_chunks = sent_per_w // chunk
    n_main = (n_chunks // NBUF) * NBUF
    assert n_chunks - n_main < NBUF and n_chunks > NBUF

    mesh = plsc.VectorSubcoreMesh(core_axis_name="c", subcore_axis_name="s")

    @pl.kernel(
        out_type=jax.ShapeDtypeStruct((SENT, D), jnp.float32),
        mesh=mesh,
        compiler_params=pltpu.CompilerParams(use_tc_tiling_on_sc=False),
        scratch_types=[
            pltpu.VMEM((NBUF, ipc), jnp.int32),
            pltpu.VMEM((NBUF, ipc, D), jnp.float32),
            pltpu.VMEM((chunk, D), jnp.float32),
            pltpu.VMEM((L, D), jnp.float32),
        ] + [pltpu.SemaphoreType.DMA] * (2 * NBUF),
    )
    def sc_call(table_hbm, idx_hbm, mask_hbm, out_hbm,
                idx_v, rows_v, out_v, mask_v, *sems):
        wid = lax.axis_index("s") * NC + lax.axis_index("c")
        pltpu.sync_copy(mask_hbm, mask_v)
        sent_base = wid * sent_per_w
        idx_base = sent_base * L
        sems_g = sems[0:NBUF]
        sems_i = sems[NBUF:2 * NBUF]

        def idx_src(c):
            return idx_hbm.at[pl.ds(idx_base + c * ipc, ipc)]

        def out_dst(c):
            return out_hbm.at[pl.ds(sent_base + c * chunk, chunk)]

        def fire_gathers(b):
            for j in range(n_streams):
                js = pl.ds(j * IDX_PER_STREAM, IDX_PER_STREAM)
                pltpu.async_copy(table_hbm.at[idx_v.at[b, js]],
                                 rows_v.at[b, js], sems_g[b])

        def drain_gathers(b):
            pltpu.make_async_copy(table_hbm.at[idx_v.at[b]],
                                  rows_v.at[b], sems_g[b]).wait()

        def compute(c, b):
            for dc in range(D // 16):
                dsl = pl.ds(dc * 16, 16)
                m = [mask_v[l, dsl] for l in range(L)]

                @pl.loop(0, chunk, unroll=2)
                def _sent(s):
                    base = s * L
                    acc = rows_v[b, base, dsl] * m[0]
                    for l in range(1, L):
                        acc = acc + rows_v[b, base + l, dsl] * m[l]
                    out_v[s, dsl] = acc

            pltpu.sync_copy(out_v, out_dst(c))

        for b in range(NBUF):
            pltpu.sync_copy(idx_src(b), idx_v.at[b])
            fire_gathers(b)

        @pl.loop(0, n_main, step=NBUF)
        def _chunks(c):
            for b in range(NBUF):
                cc = c + b
                nxt = cc + NBUF
                drain_gathers(b)

                @pl.when(nxt < n_chunks)
                def _prefetch_idx():
                    pltpu.async_copy(idx_src(nxt), idx_v.at[b], sems_i[b])

                compute(cc, b)

                @pl.when(nxt < n_chunks)
                def _fire_next():
                    pltpu.make_async_copy(idx_src(nxt), idx_v.at[b],
                                          sems_i[b]).wait()
                    fire_gathers(b)

        for cc in range(n_main, n_chunks):
            b = cc % NBUF
            drain_gathers(b)
            compute(cc, b)

    return sc_call


def kernel(stories, table, mask):
    B, S, L = stories.shape
    V, D = table.shape
    idx_flat = stories.astype(jnp.int32).reshape(-1)
    sc_call = _make_sc_call(B, S, L, D, V)
    out = sc_call(table, idx_flat, mask.astype(jnp.float32))
    return out.reshape(B, S, D)


# speedup vs baseline: 1.0131x; 1.0131x over previous
import jax
import jax.numpy as jnp
from jax import lax
from jax.experimental import pallas as pl
from jax.experimental.pallas import tpu as pltpu
from jax.experimental.pallas import tpu_sc as plsc

NC = 2
NS = 16
NW = NC * NS

IDX_PER_STREAM = 128
CHUNK = 32
NBUF = 2


def _gcd(a, b):
    while b:
        a, b = b, a % b
    return a


def _make_sc_call(B, S, L, D, V):
    SENT = B * S
    assert SENT % NW == 0
    sent_per_w = SENT // NW
    chunk = CHUNK
    ipc = chunk * L
    assert ipc % IDX_PER_STREAM == 0 and IDX_PER_STREAM % 8 == 0
    n_streams = ipc // IDX_PER_STREAM
    assert sent_per_w % (chunk * NBUF) == 0
    n_chunks = sent_per_w // chunk

    mesh = plsc.VectorSubcoreMesh(core_axis_name="c", subcore_axis_name="s")

    @pl.kernel(
        out_type=jax.ShapeDtypeStruct((SENT, D), jnp.float32),
        mesh=mesh,
        compiler_params=pltpu.CompilerParams(use_tc_tiling_on_sc=False),
        scratch_types=[
            pltpu.VMEM((NBUF, ipc), jnp.int32),
            pltpu.VMEM((NBUF, ipc, D), jnp.float32),
            pltpu.VMEM((NBUF, chunk, D), jnp.float32),
            pltpu.VMEM((L, D), jnp.float32),
        ] + [pltpu.SemaphoreType.DMA] * (3 * NBUF),
    )
    def sc_call(table_hbm, idx_hbm, mask_hbm, out_hbm,
                idx_v, rows_v, out_v, mask_v, *sems):
        wid = lax.axis_index("s") * NC + lax.axis_index("c")
        pltpu.sync_copy(mask_hbm, mask_v)
        sent_base = wid * sent_per_w
        idx_base = sent_base * L
        sems_g = sems[0:NBUF]
        sems_i = sems[NBUF:2 * NBUF]
        sems_o = sems[2 * NBUF:3 * NBUF]

        def idx_src(c):
            return idx_hbm.at[pl.ds(idx_base + c * ipc, ipc)]

        def out_dst(c):
            return out_hbm.at[pl.ds(sent_base + c * chunk, chunk)]

        def fire_gathers(b):
            for j in range(n_streams):
                js = pl.ds(j * IDX_PER_STREAM, IDX_PER_STREAM)
                pltpu.async_copy(table_hbm.at[idx_v.at[b, js]],
                                 rows_v.at[b, js], sems_g[b])

        def drain_gathers(b):
            pltpu.make_async_copy(table_hbm.at[idx_v.at[b]],
                                  rows_v.at[b], sems_g[b]).wait()

        def compute(c, b):
            for dc in range(D // 16):
                dsl = pl.ds(dc * 16, 16)
                m = [mask_v[l, dsl] for l in range(L)]

                @pl.loop(0, chunk, unroll=2)
                def _sent(s):
                    base = s * L
                    acc = rows_v[b, base, dsl] * m[0]
                    for l in range(1, L):
                        acc = acc + rows_v[b, base + l, dsl] * m[l]
                    out_v[b, s, dsl] = acc

            pltpu.async_copy(out_v.at[b], out_dst(c), sems_o[b])

        for b in range(NBUF):
            pltpu.sync_copy(idx_src(b), idx_v.at[b])
            fire_gathers(b)

        @pl.loop(0, n_chunks, step=NBUF)
        def _chunks(c):
            for b in range(NBUF):
                cc = c + b
                nxt = cc + NBUF
                drain_gathers(b)

                @pl.when(nxt < n_chunks)
                def _prefetch_idx():
                    pltpu.async_copy(idx_src(nxt), idx_v.at[b], sems_i[b])

                @pl.when(cc >= NBUF)
                def _drain_out():
                    pltpu.make_async_copy(out_v.at[b], out_dst(cc - NBUF),
                                          sems_o[b]).wait()

                compute(cc, b)

                @pl.when(nxt < n_chunks)
                def _fire_next():
                    pltpu.make_async_copy(idx_src(nxt), idx_v.at[b],
                                          sems_i[b]).wait()
                    fire_gathers(b)

        for b in range(NBUF):
            pltpu.make_async_copy(out_v.at[b], out_dst(n_chunks - NBUF + b),
                                  sems_o[b]).wait()

    return sc_call


def kernel(stories, table, mask):
    B, S, L = stories.shape
    V, D = table.shape
    idx_flat = stories.astype(jnp.int32).reshape(-1)
    sc_call = _make_sc_call(B, S, L, D, V)
    out = sc_call(table, idx_flat, mask.astype(jnp.float32))
    return out.reshape(B, S, D)
